# baseline (speedup 1.0000x reference)
import jax
import jax.numpy as jnp
from jax.experimental import pallas as pl
from jax.experimental.pallas import tpu as pltpu

_B, _C, _H, _W = 64, 3, 224, 224
_ROWS = _C * _H * _W // 128


def _scale_body(p_ref, m_ref, x_ref, o_ref):
    i = pl.program_id(0)
    scale = jnp.where(p_ref[i] != 0, m_ref[i], jnp.float32(1.0))
    o_ref[...] = x_ref[...] * scale


def kernel(input, probs, magnitudes):
    x = input.reshape(_B, _ROWS, 128)
    p = probs.astype(jnp.int32)
    out = pl.pallas_call(
        _scale_body,
        grid=(_B,),
        in_specs=[
            pl.BlockSpec(memory_space=pltpu.SMEM),
            pl.BlockSpec(memory_space=pltpu.SMEM),
            pl.BlockSpec((1, _ROWS, 128), lambda i: (i, 0, 0)),
        ],
        out_specs=pl.BlockSpec((1, _ROWS, 128), lambda i: (i, 0, 0)),
        out_shape=jax.ShapeDtypeStruct((_B, _ROWS, 128), jnp.float32),
    )(p, magnitudes, x)
    return out.reshape(_B, _C, _H, _W)


# speedup vs baseline: 2.8208x; 2.8208x over previous
import jax
import jax.numpy as jnp
from jax.experimental import pallas as pl
from jax.experimental.pallas import tpu as pltpu

_B, _C, _H, _W = 64, 3, 224, 224
_ROWS = _C * _H * _W // 128


def _scale_body(p_ref, m_ref, x_ref, o_ref):
    i = pl.program_id(0)
    scale = jnp.where(p_ref[i] != 0, m_ref[i], jnp.float32(1.0))
    o_ref[...] = x_ref[...] * scale


def kernel(input, probs, magnitudes):
    p = probs.astype(jnp.int32)
    return pl.pallas_call(
        _scale_body,
        grid=(_B,),
        in_specs=[
            pl.BlockSpec(memory_space=pltpu.SMEM),
            pl.BlockSpec(memory_space=pltpu.SMEM),
            pl.BlockSpec((1, _C, _H, _W), lambda i: (i, 0, 0, 0)),
        ],
        out_specs=pl.BlockSpec((1, _C, _H, _W), lambda i: (i, 0, 0, 0)),
        out_shape=jax.ShapeDtypeStruct((_B, _C, _H, _W), jnp.float32),
    )(p, magnitudes, input)


# speedup vs baseline: 4.8555x; 1.7213x over previous
import jax
import jax.numpy as jnp
from jax.experimental import pallas as pl
from jax.experimental.pallas import tpu as pltpu

_B, _C, _H, _W = 64, 3, 224, 224
_ROWS = _C * _H * _W // 128


_BK = 4


def _scale_body(p_ref, m_ref, x_ref, o_ref):
    i = pl.program_id(0)
    for j in range(_BK):
        b = i * _BK + j
        scale = jnp.where(p_ref[b] != 0, m_ref[b], jnp.float32(1.0))
        o_ref[j] = x_ref[j] * scale


def kernel(input, probs, magnitudes):
    p = probs.astype(jnp.int32)
    return pl.pallas_call(
        _scale_body,
        grid=(_B // _BK,),
        in_specs=[
            pl.BlockSpec(memory_space=pltpu.SMEM),
            pl.BlockSpec(memory_space=pltpu.SMEM),
            pl.BlockSpec((_BK, _C, _H, _W), lambda i: (i, 0, 0, 0)),
        ],
        out_specs=pl.BlockSpec((_BK, _C, _H, _W), lambda i: (i, 0, 0, 0)),
        out_shape=jax.ShapeDtypeStruct((_B, _C, _H, _W), jnp.float32),
    )(p, magnitudes, input)


# speedup vs baseline: 5.1322x; 1.0570x over previous
import jax
import jax.numpy as jnp
from jax.experimental import pallas as pl
from jax.experimental.pallas import tpu as pltpu

_B, _C, _H, _W = 64, 3, 224, 224
_ROWS = _C * _H * _W // 128


_BK = 8


def _scale_body(p_ref, m_ref, x_ref, o_ref):
    i = pl.program_id(0)
    for j in range(_BK):
        b = i * _BK + j
        scale = jnp.where(p_ref[b] != 0, m_ref[b], jnp.float32(1.0))
        o_ref[j] = x_ref[j] * scale


def kernel(input, probs, magnitudes):
    p = probs.astype(jnp.int32)
    return pl.pallas_call(
        _scale_body,
        grid=(_B // _BK,),
        in_specs=[
            pl.BlockSpec(memory_space=pltpu.SMEM),
            pl.BlockSpec(memory_space=pltpu.SMEM),
            pl.BlockSpec((_BK, _C, _H, _W), lambda i: (i, 0, 0, 0)),
        ],
        out_specs=pl.BlockSpec((_BK, _C, _H, _W), lambda i: (i, 0, 0, 0)),
        out_shape=jax.ShapeDtypeStruct((_B, _C, _H, _W), jnp.float32),
    )(p, magnitudes, input)


# speedup vs baseline: 5.3257x; 1.0377x over previous
import jax
import jax.numpy as jnp
from jax.experimental import pallas as pl
from jax.experimental.pallas import tpu as pltpu

_B, _C, _H, _W = 64, 3, 224, 224
_ROWS = _C * _H * _W // 128


_BK = 16


def _scale_body(p_ref, m_ref, x_ref, o_ref):
    i = pl.program_id(0)
    for j in range(_BK):
        b = i * _BK + j
        scale = jnp.where(p_ref[b] != 0, m_ref[b], jnp.float32(1.0))
        o_ref[j] = x_ref[j] * scale


def kernel(input, probs, magnitudes):
    p = probs.astype(jnp.int32)
    return pl.pallas_call(
        _scale_body,
        grid=(_B // _BK,),
        in_specs=[
            pl.BlockSpec(memory_space=pltpu.SMEM),
            pl.BlockSpec(memory_space=pltpu.SMEM),
            pl.BlockSpec((_BK, _C, _H, _W), lambda i: (i, 0, 0, 0)),
        ],
        out_specs=pl.BlockSpec((_BK, _C, _H, _W), lambda i: (i, 0, 0, 0)),
        out_shape=jax.ShapeDtypeStruct((_B, _C, _H, _W), jnp.float32),
    )(p, magnitudes, input)
